# baseline (speedup 1.0000x reference)
import functools

import jax
import jax.numpy as jnp
from jax import lax
from jax.experimental import pallas as pl
from jax.experimental.pallas import tpu as pltpu
from jax.experimental.pallas import tpu_sc as plsc

HIDDEN = 1024
NUM_WORKERS = 32
CHUNK = 16
NBUF = 4
IN_FLIGHT = NBUF - 1


def _make_lookup(b_total: int):
    b_per_w = b_total // NUM_WORKERS
    n_chunk = b_per_w // CHUNK

    mesh = plsc.VectorSubcoreMesh(core_axis_name="c", subcore_axis_name="s")

    @functools.partial(
        pl.kernel,
        mesh=mesh,
        out_type=jax.ShapeDtypeStruct((b_total, HIDDEN), jnp.float32),
        scratch_types=[
            pltpu.VMEM((n_chunk, CHUNK), jnp.int32),
            pltpu.VMEM((NBUF, CHUNK, HIDDEN), jnp.float32),
        ]
        + [pltpu.SemaphoreType.DMA] * (2 * NBUF),
    )
    def lookup(idx_hbm, table_hbm, out_hbm, idx_v, rows_v, *sems):
        wid = lax.axis_index("s") * 2 + lax.axis_index("c")
        base = wid * b_per_w
        gsem = sems[:NBUF]
        ssem = sems[NBUF:]
        pltpu.sync_copy(idx_hbm.at[wid], idx_v)

        def gather(c):
            return pltpu.async_copy(
                table_hbm.at[idx_v.at[c]], rows_v.at[c % NBUF],
                gsem[c % NBUF])

        def scatter(c):
            return pltpu.async_copy(
                rows_v.at[c % NBUF],
                out_hbm.at[pl.ds(base + c * CHUNK, CHUNK)],
                ssem[c % NBUF])

        gathers = [None] * n_chunk
        scatters = [None] * n_chunk
        for k in range(IN_FLIGHT):
            gathers[k] = gather(k)
        for c in range(n_chunk):
            nxt = c + IN_FLIGHT
            if nxt < n_chunk:
                if nxt >= NBUF:
                    scatters[nxt - NBUF].wait()
                gathers[nxt] = gather(nxt)
            gathers[c].wait()
            scatters[c] = scatter(c)
        for c in range(max(0, n_chunk - NBUF), n_chunk):
            scatters[c].wait()

    return lookup


def kernel(input_ids, embed_tokens_weight):
    b, s = input_ids.shape
    b_total = b * s
    ids = input_ids.astype(jnp.int32).reshape(
        NUM_WORKERS, b_total // (NUM_WORKERS * CHUNK), CHUNK)
    out = _make_lookup(b_total)(ids, embed_tokens_weight)
    return out.reshape(b, s, HIDDEN)


# speedup vs baseline: 1.0111x; 1.0111x over previous
import functools

import jax
import jax.numpy as jnp
from jax import lax
from jax.experimental import pallas as pl
from jax.experimental.pallas import tpu as pltpu
from jax.experimental.pallas import tpu_sc as plsc

HIDDEN = 1024
NUM_WORKERS = 32
CHUNK = 16
NBUF = 6
IN_FLIGHT = NBUF - 1


def _make_lookup(b_total: int):
    b_per_w = b_total // NUM_WORKERS
    n_chunk = b_per_w // CHUNK

    mesh = plsc.VectorSubcoreMesh(core_axis_name="c", subcore_axis_name="s")

    @functools.partial(
        pl.kernel,
        mesh=mesh,
        out_type=jax.ShapeDtypeStruct((b_total, HIDDEN), jnp.float32),
        scratch_types=[
            pltpu.VMEM((n_chunk, CHUNK), jnp.int32),
            pltpu.VMEM((NBUF, CHUNK, HIDDEN), jnp.float32),
        ]
        + [pltpu.SemaphoreType.DMA] * (2 * NBUF),
    )
    def lookup(idx_hbm, table_hbm, out_hbm, idx_v, rows_v, *sems):
        wid = lax.axis_index("s") * 2 + lax.axis_index("c")
        base = wid * b_per_w
        gsem = sems[:NBUF]
        ssem = sems[NBUF:]
        pltpu.sync_copy(idx_hbm.at[wid], idx_v)

        def gather(c):
            return pltpu.async_copy(
                table_hbm.at[idx_v.at[c]], rows_v.at[c % NBUF],
                gsem[c % NBUF])

        def scatter(c):
            return pltpu.async_copy(
                rows_v.at[c % NBUF],
                out_hbm.at[pl.ds(base + c * CHUNK, CHUNK)],
                ssem[c % NBUF])

        gathers = [None] * n_chunk
        scatters = [None] * n_chunk
        for k in range(IN_FLIGHT):
            gathers[k] = gather(k)
        for c in range(n_chunk):
            nxt = c + IN_FLIGHT
            if nxt < n_chunk:
                if nxt >= NBUF:
                    scatters[nxt - NBUF].wait()
                gathers[nxt] = gather(nxt)
            gathers[c].wait()
            scatters[c] = scatter(c)
        for c in range(max(0, n_chunk - NBUF), n_chunk):
            scatters[c].wait()

    return lookup


def kernel(input_ids, embed_tokens_weight):
    b, s = input_ids.shape
    b_total = b * s
    ids = input_ids.astype(jnp.int32).reshape(
        NUM_WORKERS, b_total // (NUM_WORKERS * CHUNK), CHUNK)
    out = _make_lookup(b_total)(ids, embed_tokens_weight)
    return out.reshape(b, s, HIDDEN)
